# baseline (speedup 1.0000x reference)
import functools

import jax
import jax.numpy as jnp
from jax import lax
from jax.experimental import pallas as pl
from jax.experimental.pallas import tpu as pltpu
from jax.experimental.pallas import tpu_sc as plsc

_NUM_TYPES = 100
_NUM_REPS = 3
_EMBED = 64
_ROW = (_NUM_TYPES + _NUM_REPS) * _EMBED
_PAIR = 2 * _ROW

_NC = 2
_NS = 16
_NW = _NC * _NS

_CHUNK = 8


def _bcast_sc(table2, batch):
    pairs = batch // 2
    p_per_w = pairs // _NW
    n_chunks = p_per_w // _CHUNK
    mesh = plsc.VectorSubcoreMesh(core_axis_name="c", subcore_axis_name="s")

    @functools.partial(
        pl.kernel,
        mesh=mesh,
        out_type=jax.ShapeDtypeStruct((pairs, _PAIR), jnp.float32),
        scratch_types=[
            pltpu.VMEM((_CHUNK, _PAIR), jnp.float32),
            pltpu.SemaphoreType.DMA,
            pltpu.SemaphoreType.DMA,
        ],
    )
    def body(table2_hbm, out_hbm, buf, load_sem, store_sem):
        wid = lax.axis_index("s") * _NC + lax.axis_index("c")
        base = wid * p_per_w
        loads = [
            pltpu.async_copy(table2_hbm, buf.at[i], load_sem)
            for i in range(_CHUNK)
        ]
        for c in loads:
            c.wait()
        stores = [
            pltpu.async_copy(
                buf, out_hbm.at[pl.ds(base + j * _CHUNK, _CHUNK)], store_sem
            )
            for j in range(n_chunks)
        ]
        for c in stores:
            c.wait()

    return body(table2)


def kernel(features, type_table, rep_table):
    batch = features.shape[0]
    row = jnp.concatenate(
        [type_table.reshape(-1), rep_table.reshape(-1)]
    )
    table2 = jnp.concatenate([row, row])
    out = _bcast_sc(table2, batch)
    return out.reshape(batch, _NUM_TYPES + _NUM_REPS, _EMBED)


# speedup vs baseline: 7.4443x; 7.4443x over previous
import functools

import jax
import jax.numpy as jnp
from jax import lax
from jax.experimental import pallas as pl
from jax.experimental.pallas import tpu as pltpu
from jax.experimental.pallas import tpu_sc as plsc

_NUM_TYPES = 100
_NUM_REPS = 3
_EMBED = 64
_NROWS = _NUM_TYPES + _NUM_REPS
_FLAT = _NROWS * _EMBED

_NC = 2
_NS = 16
_NW = _NC * _NS

_RPT = 208
_LAST = _FLAT - _RPT * (_NW - 1)
_CW = 512
_LANES = 16


def _bcast_sc(table, batch):
    n_cchunks = batch // _CW
    mesh = plsc.VectorSubcoreMesh(core_axis_name="c", subcore_axis_name="s")

    @functools.partial(
        pl.kernel,
        mesh=mesh,
        out_type=jax.ShapeDtypeStruct((_FLAT, batch), jnp.float32),
        scratch_types=[
            pltpu.VMEM((_RPT * _NW + _LANES,), jnp.float32),
            pltpu.VMEM((_RPT, _CW), jnp.float32),
            pltpu.SemaphoreType.DMA,
            pltpu.SemaphoreType.DMA,
        ],
    )
    def body(table_hbm, out_hbm, tbl_v, buf, local_sem, store_sem):
        wid = lax.axis_index("s") * _NC + lax.axis_index("c")
        row0 = wid * _RPT
        pltpu.async_copy(table_hbm, tbl_v.at[pl.ds(0, _FLAT)], local_sem).wait()

        def build_row(i, _):
            v = tbl_v[pl.ds(row0 + i, _LANES)]
            splat = jnp.full((_LANES,), v[0], dtype=jnp.float32)
            for k in range(_CW // _LANES):
                buf[i, pl.ds(k * _LANES, _LANES)] = splat
            return 0

        lax.fori_loop(0, _RPT, build_row, 0)

        @pl.when(wid < _NW - 1)
        def _():
            stores = [
                pltpu.async_copy(
                    buf,
                    out_hbm.at[pl.ds(row0, _RPT), pl.ds(cc * _CW, _CW)],
                    store_sem,
                )
                for cc in range(n_cchunks)
            ]
            for s in stores:
                s.wait()

        @pl.when(wid == _NW - 1)
        def _():
            stores = [
                pltpu.async_copy(
                    buf.at[pl.ds(0, _LAST)],
                    out_hbm.at[pl.ds(row0, _LAST), pl.ds(cc * _CW, _CW)],
                    store_sem,
                )
                for cc in range(n_cchunks)
            ]
            for s in stores:
                s.wait()

    return body(table)


def kernel(features, type_table, rep_table):
    batch = features.shape[0]
    table = jnp.concatenate(
        [type_table.reshape(-1), rep_table.reshape(-1)]
    )
    out = _bcast_sc(table, batch)
    return jnp.transpose(
        out.reshape(_NROWS, _EMBED, batch), (2, 0, 1)
    )


# speedup vs baseline: 7.4664x; 1.0030x over previous
import functools

import jax
import jax.numpy as jnp
from jax import lax
from jax.experimental import pallas as pl
from jax.experimental.pallas import tpu as pltpu
from jax.experimental.pallas import tpu_sc as plsc

_NUM_TYPES = 100
_NUM_REPS = 3
_EMBED = 64
_NROWS = _NUM_TYPES + _NUM_REPS
_FLAT = _NROWS * _EMBED

_NC = 2
_NS = 16
_NW = _NC * _NS

_RPT = 208
_LAST = _FLAT - _RPT * (_NW - 1)
_CW = 512
_LANES = 16


def _bcast_sc(table, batch):
    n_cchunks = batch // _CW
    mesh = plsc.VectorSubcoreMesh(core_axis_name="c", subcore_axis_name="s")

    @functools.partial(
        pl.kernel,
        mesh=mesh,
        out_type=jax.ShapeDtypeStruct((_FLAT, batch), jnp.float32),
        scratch_types=[
            pltpu.VMEM((_RPT * _NW + _LANES,), jnp.float32),
            pltpu.VMEM((_RPT, _CW), jnp.float32),
            pltpu.SemaphoreType.DMA,
            pltpu.SemaphoreType.DMA,
        ],
    )
    def body(table_hbm, out_hbm, tbl_v, buf, local_sem, store_sem):
        wid = lax.axis_index("s") * _NC + lax.axis_index("c")
        row0 = wid * _RPT
        pltpu.async_copy(table_hbm, tbl_v.at[pl.ds(0, _FLAT)], local_sem).wait()

        def build_rows(lo, n):
            def build_row(i, _):
                v = tbl_v[pl.ds(row0 + i, _LANES)]
                splat = jnp.full((_LANES,), v[0], dtype=jnp.float32)
                for k in range(_CW // _LANES):
                    buf[i, pl.ds(k * _LANES, _LANES)] = splat
                return 0

            lax.fori_loop(lo, lo + n, build_row, 0)

        def fire(lo, n):
            return [
                pltpu.async_copy(
                    buf.at[pl.ds(lo, n)],
                    out_hbm.at[pl.ds(row0 + lo, n), pl.ds(cc * _CW, _CW)],
                    store_sem,
                )
                for cc in range(n_cchunks)
            ]

        _H = _RPT // 2
        build_rows(0, _H)

        @pl.when(wid < _NW - 1)
        def _():
            s1 = fire(0, _H)
            build_rows(_H, _H)
            s2 = fire(_H, _H)
            for s in s1 + s2:
                s.wait()

        @pl.when(wid == _NW - 1)
        def _():
            s1 = fire(0, _H)
            build_rows(_H, _LAST - _H)
            s2 = fire(_H, _LAST - _H)
            for s in s1 + s2:
                s.wait()

    return body(table)


def kernel(features, type_table, rep_table):
    batch = features.shape[0]
    table = jnp.concatenate(
        [type_table.reshape(-1), rep_table.reshape(-1)]
    )
    out = _bcast_sc(table, batch)
    return jnp.transpose(
        out.reshape(_NROWS, _EMBED, batch), (2, 0, 1)
    )
